# baseline (speedup 1.0000x reference)
import functools

import jax
import jax.numpy as jnp
from jax import lax
from jax.experimental import pallas as pl
from jax.experimental.pallas import tpu as pltpu
from jax.experimental.pallas import tpu_sc as plsc

_M = 16384
_D = 256
_K = 8192

_TM = 256
_TK = 2048
_NM = _M // _TM
_NK = _K // _TK

_NC = 2
_NS = 16
_NW = _NC * _NS
_CH = 128


def _assign_body(x2_ref, x_ref, ct_ref, c2_ref, iw_ref, idx_ref, val_ref):
    xs = x_ref[...] * 2.0
    mm = jnp.dot(xs, ct_ref[...], preferred_element_type=jnp.float32)
    t = (x2_ref[...] - mm) + c2_ref[...]
    bmin = jnp.min(t, axis=1, keepdims=True)
    val_ref[...] = bmin
    onehot = jnp.where(t == bmin, 1.0, 0.0)
    sc = jnp.dot(onehot, iw_ref[...], preferred_element_type=jnp.float32)
    sumidx = sc[:, 0:1]
    cnt = sc[:, 1:2]
    has_tie = jnp.max(cnt) > 1.5

    @pl.when(jnp.logical_not(has_tie))
    def _no_tie():
        idx_ref[...] = sumidx.astype(jnp.int32)

    @pl.when(has_tie)
    def _tie():
        idsf = lax.broadcasted_iota(jnp.int32, (_TM, _K), 1).astype(
            jnp.float32)
        barg = jnp.min(jnp.where(t == bmin, idsf, float(_K)),
                       axis=1, keepdims=True)
        idx_ref[...] = barg.astype(jnp.int32)


_assign_call = pl.pallas_call(
    _assign_body,
    grid=(_NM,),
    in_specs=[
        pl.BlockSpec((_TM, 1), lambda m: (m, 0)),
        pl.BlockSpec((_TM, _D), lambda m: (m, 0)),
        pl.BlockSpec((_D, _K), lambda m: (0, 0)),
        pl.BlockSpec((1, _K), lambda m: (0, 0)),
        pl.BlockSpec((_K, 2), lambda m: (0, 0)),
    ],
    out_specs=[
        pl.BlockSpec((_TM, 1), lambda m: (m, 0)),
        pl.BlockSpec((_TM, 1), lambda m: (m, 0)),
    ],
    out_shape=[
        jax.ShapeDtypeStruct((_M, 1), jnp.int32),
        jax.ShapeDtypeStruct((_M, 1), jnp.float32),
    ],
)


_BPW = _M // _NW


def _gather_body(table_hbm, idx_hbm, out_hbm, idx_v, rows_v, sem):
    wid = lax.axis_index("s") * _NC + lax.axis_index("c")
    base = wid * _BPW
    for j in range(_BPW // _CH):
        off = base + j * _CH
        pltpu.sync_copy(idx_hbm.at[pl.ds(off, _CH)], idx_v)
        pltpu.async_copy(table_hbm.at[idx_v], rows_v, sem).wait()
        pltpu.sync_copy(rows_v, out_hbm.at[pl.ds(off, _CH)])


@functools.lru_cache(maxsize=1)
def _make_gather_call():
    return functools.partial(
        pl.kernel,
        mesh=plsc.VectorSubcoreMesh(core_axis_name="c", subcore_axis_name="s"),
        out_type=jax.ShapeDtypeStruct((_M, _D), jnp.float32),
        scratch_types=[
            pltpu.VMEM((_CH,), jnp.int32),
            pltpu.VMEM((_CH, _D), jnp.float32),
            pltpu.SemaphoreType.DMA,
        ],
    )(_gather_body)


def kernel(x, codebook):
    orig_shape = x.shape
    flatten = x.reshape(-1, orig_shape[-1])
    x2 = jnp.sum(flatten ** 2, axis=1, keepdims=True)
    c2 = jnp.sum(codebook ** 2, axis=1)[None, :]
    ct = codebook.T

    iw = jnp.stack([jnp.arange(_K, dtype=jnp.float32),
                    jnp.ones((_K,), jnp.float32)], axis=1)
    idx2d, val2d = _assign_call(x2, flatten, ct, c2, iw)
    embed_ind = idx2d.reshape(-1)
    commit_loss = jnp.sum(val2d) / (_M * _D)

    quantize = _make_gather_call()(codebook, embed_ind)
    quantize_st = quantize.reshape(orig_shape)
    return quantize_st, embed_ind.reshape(orig_shape[:-1]), commit_loss


# speedup vs baseline: 1.8421x; 1.8421x over previous
import functools

import jax
import jax.numpy as jnp
from jax import lax
from jax.experimental import pallas as pl
from jax.experimental.pallas import tpu as pltpu
from jax.experimental.pallas import tpu_sc as plsc

_M = 16384
_D = 256
_K = 8192

_TM = 512
_TK = 2048
_NM = _M // _TM
_NK = _K // _TK

_NC = 2
_NS = 16
_NW = _NC * _NS
_CH = 128


def _assign_body(x2_ref, x_ref, ct_ref, c2_ref, ids_ref, idx_ref, val_ref):
    xs = x_ref[...] * 2.0
    mm = jnp.dot(xs, ct_ref[...], preferred_element_type=jnp.float32)
    t = (x2_ref[...] - mm) + c2_ref[...]
    bmin = jnp.min(t, axis=1, keepdims=True)
    barg = jnp.min(jnp.where(t == bmin, ids_ref[...], float(_K)),
                   axis=1, keepdims=True)
    idx_ref[...] = barg.astype(jnp.int32)
    val_ref[...] = bmin


_assign_call = pl.pallas_call(
    _assign_body,
    grid=(_NM,),
    in_specs=[
        pl.BlockSpec((_TM, 1), lambda m: (m, 0)),
        pl.BlockSpec((_TM, _D), lambda m: (m, 0)),
        pl.BlockSpec((_D, _K), lambda m: (0, 0)),
        pl.BlockSpec((1, _K), lambda m: (0, 0)),
        pl.BlockSpec((1, _K), lambda m: (0, 0)),
    ],
    out_specs=[
        pl.BlockSpec((_TM, 1), lambda m: (m, 0)),
        pl.BlockSpec((_TM, 1), lambda m: (m, 0)),
    ],
    out_shape=[
        jax.ShapeDtypeStruct((_M, 1), jnp.int32),
        jax.ShapeDtypeStruct((_M, 1), jnp.float32),
    ],
)


_BPW = _M // _NW


def _gather_body(table_hbm, idx_hbm, out_hbm, idx_v, rows_v, sem):
    wid = lax.axis_index("s") * _NC + lax.axis_index("c")
    base = wid * _BPW
    for j in range(_BPW // _CH):
        off = base + j * _CH
        pltpu.sync_copy(idx_hbm.at[pl.ds(off, _CH)], idx_v)
        pltpu.async_copy(table_hbm.at[idx_v], rows_v, sem).wait()
        pltpu.sync_copy(rows_v, out_hbm.at[pl.ds(off, _CH)])


@functools.lru_cache(maxsize=1)
def _make_gather_call():
    return functools.partial(
        pl.kernel,
        mesh=plsc.VectorSubcoreMesh(core_axis_name="c", subcore_axis_name="s"),
        out_type=jax.ShapeDtypeStruct((_M, _D), jnp.float32),
        scratch_types=[
            pltpu.VMEM((_CH,), jnp.int32),
            pltpu.VMEM((_CH, _D), jnp.float32),
            pltpu.SemaphoreType.DMA,
        ],
    )(_gather_body)


def kernel(x, codebook):
    orig_shape = x.shape
    flatten = x.reshape(-1, orig_shape[-1])
    x2 = jnp.sum(flatten ** 2, axis=1, keepdims=True)
    c2 = jnp.sum(codebook ** 2, axis=1)[None, :]
    ct = codebook.T

    ids = jnp.arange(_K, dtype=jnp.float32)[None, :]
    idx2d, val2d = _assign_call(x2, flatten, ct, c2, ids)
    embed_ind = idx2d.reshape(-1)
    commit_loss = jnp.sum(val2d) / (_M * _D)

    quantize = _make_gather_call()(codebook, embed_ind)
    quantize_st = quantize.reshape(orig_shape)
    return quantize_st, embed_ind.reshape(orig_shape[:-1]), commit_loss


# speedup vs baseline: 1.8853x; 1.0235x over previous
import functools

import jax
import jax.numpy as jnp
from jax import lax
from jax.experimental import pallas as pl
from jax.experimental.pallas import tpu as pltpu
from jax.experimental.pallas import tpu_sc as plsc

_M = 16384
_D = 256
_K = 8192

_TM = 1024
_TK = 2048
_NM = _M // _TM
_NK = _K // _TK

_NC = 2
_NS = 16
_NW = _NC * _NS
_CH = 128


def _assign_body(x2_ref, x_ref, ct_ref, c2_ref, ids_ref, idx_ref, val_ref):
    xs = x_ref[...] * 2.0
    mm = jnp.dot(xs, ct_ref[...], preferred_element_type=jnp.float32)
    t = (x2_ref[...] - mm) + c2_ref[...]
    bmin = jnp.min(t, axis=1, keepdims=True)
    barg = jnp.min(jnp.where(t == bmin, ids_ref[...], float(_K)),
                   axis=1, keepdims=True)
    idx_ref[...] = barg.astype(jnp.int32)
    val_ref[...] = bmin


_assign_call = pl.pallas_call(
    _assign_body,
    grid=(_NM,),
    in_specs=[
        pl.BlockSpec((_TM, 1), lambda m: (m, 0)),
        pl.BlockSpec((_TM, _D), lambda m: (m, 0)),
        pl.BlockSpec((_D, _K), lambda m: (0, 0)),
        pl.BlockSpec((1, _K), lambda m: (0, 0)),
        pl.BlockSpec((1, _K), lambda m: (0, 0)),
    ],
    out_specs=[
        pl.BlockSpec((_TM, 1), lambda m: (m, 0)),
        pl.BlockSpec((_TM, 1), lambda m: (m, 0)),
    ],
    out_shape=[
        jax.ShapeDtypeStruct((_M, 1), jnp.int32),
        jax.ShapeDtypeStruct((_M, 1), jnp.float32),
    ],
)


_BPW = _M // _NW


def _gather_body(table_hbm, idx_hbm, out_hbm, idx_v, rows_v, sem):
    wid = lax.axis_index("s") * _NC + lax.axis_index("c")
    base = wid * _BPW
    for j in range(_BPW // _CH):
        off = base + j * _CH
        pltpu.sync_copy(idx_hbm.at[pl.ds(off, _CH)], idx_v)
        pltpu.async_copy(table_hbm.at[idx_v], rows_v, sem).wait()
        pltpu.sync_copy(rows_v, out_hbm.at[pl.ds(off, _CH)])


@functools.lru_cache(maxsize=1)
def _make_gather_call():
    return functools.partial(
        pl.kernel,
        mesh=plsc.VectorSubcoreMesh(core_axis_name="c", subcore_axis_name="s"),
        out_type=jax.ShapeDtypeStruct((_M, _D), jnp.float32),
        scratch_types=[
            pltpu.VMEM((_CH,), jnp.int32),
            pltpu.VMEM((_CH, _D), jnp.float32),
            pltpu.SemaphoreType.DMA,
        ],
    )(_gather_body)


def kernel(x, codebook):
    orig_shape = x.shape
    flatten = x.reshape(-1, orig_shape[-1])
    x2 = jnp.sum(flatten ** 2, axis=1, keepdims=True)
    c2 = jnp.sum(codebook ** 2, axis=1)[None, :]
    ct = codebook.T

    ids = jnp.arange(_K, dtype=jnp.float32)[None, :]
    idx2d, val2d = _assign_call(x2, flatten, ct, c2, ids)
    embed_ind = idx2d.reshape(-1)
    commit_loss = jnp.sum(val2d) / (_M * _D)

    quantize = _make_gather_call()(codebook, embed_ind)
    quantize_st = quantize.reshape(orig_shape)
    return quantize_st, embed_ind.reshape(orig_shape[:-1]), commit_loss


# speedup vs baseline: 1.9027x; 1.0092x over previous
import functools

import jax
import jax.numpy as jnp
from jax import lax
from jax.experimental import pallas as pl
from jax.experimental.pallas import tpu as pltpu
from jax.experimental.pallas import tpu_sc as plsc

_M = 16384
_D = 256
_K = 8192

_TM = 1024
_TK = 2048
_NM = _M // _TM
_NK = _K // _TK

_NC = 2
_NS = 16
_NW = _NC * _NS
_CH = 128


def _assign_body(x2_ref, x_ref, ct_ref, c2_ref, ids_ref, idx_ref, val_ref):
    xs = x_ref[...] * 2.0
    mm = jnp.dot(xs, ct_ref[...], preferred_element_type=jnp.float32)
    t = (x2_ref[...] - mm) + c2_ref[...]
    bmin = jnp.min(t, axis=1, keepdims=True)
    barg = jnp.min(jnp.where(t == bmin, ids_ref[...], float(_K)),
                   axis=1, keepdims=True)
    idx_ref[...] = barg.astype(jnp.int32)
    val_ref[...] = bmin


_assign_call = pl.pallas_call(
    _assign_body,
    grid=(_NM,),
    in_specs=[
        pl.BlockSpec((_TM, 1), lambda m: (m, 0)),
        pl.BlockSpec((_TM, _D), lambda m: (m, 0)),
        pl.BlockSpec((_D, _K), lambda m: (0, 0)),
        pl.BlockSpec((1, _K), lambda m: (0, 0)),
        pl.BlockSpec((1, _K), lambda m: (0, 0)),
    ],
    out_specs=[
        pl.BlockSpec((_TM, 1), lambda m: (m, 0)),
        pl.BlockSpec((_TM, 1), lambda m: (m, 0)),
    ],
    out_shape=[
        jax.ShapeDtypeStruct((_M, 1), jnp.int32),
        jax.ShapeDtypeStruct((_M, 1), jnp.float32),
    ],
)


_BPW = _M // _NW
_NCH = _BPW // _CH


def _gather_body(table_hbm, idx_hbm, out_hbm, idx_v,
                 buf0, buf1, sg0, sg1, so0, so1):
    wid = lax.axis_index("s") * _NC + lax.axis_index("c")
    base = wid * _NCH
    pltpu.sync_copy(idx_hbm.at[pl.ds(base, _NCH)], idx_v)
    bufs = (buf0, buf1)
    gsems = (sg0, sg1)
    osems = (so0, so1)
    gh = {}
    oh = {}
    gh[0] = pltpu.async_copy(table_hbm.at[idx_v.at[0]], buf0, sg0)
    gh[1] = pltpu.async_copy(table_hbm.at[idx_v.at[1]], buf1, sg1)
    for j in range(_NCH):
        b = j % 2
        gh[j].wait()
        oh[j] = pltpu.async_copy(
            bufs[b], out_hbm.at[pl.ds((base + j) * _CH, _CH)], osems[b])
        if j + 2 < _NCH:
            oh[j].wait()
            gh[j + 2] = pltpu.async_copy(
                table_hbm.at[idx_v.at[j + 2]], bufs[b], gsems[b])
    oh[_NCH - 2].wait()
    oh[_NCH - 1].wait()


@functools.lru_cache(maxsize=1)
def _make_gather_call():
    return functools.partial(
        pl.kernel,
        mesh=plsc.VectorSubcoreMesh(core_axis_name="c", subcore_axis_name="s"),
        out_type=jax.ShapeDtypeStruct((_M, _D), jnp.float32),
        scratch_types=[
            pltpu.VMEM((_NCH, _CH), jnp.int32),
            pltpu.VMEM((_CH, _D), jnp.float32),
            pltpu.VMEM((_CH, _D), jnp.float32),
            pltpu.SemaphoreType.DMA,
            pltpu.SemaphoreType.DMA,
            pltpu.SemaphoreType.DMA,
            pltpu.SemaphoreType.DMA,
        ],
    )(_gather_body)


def kernel(x, codebook):
    orig_shape = x.shape
    flatten = x.reshape(-1, orig_shape[-1])
    x2 = jnp.sum(flatten ** 2, axis=1, keepdims=True)
    c2 = jnp.sum(codebook ** 2, axis=1)[None, :]
    ct = codebook.T

    ids = jnp.arange(_K, dtype=jnp.float32)[None, :]
    idx2d, val2d = _assign_call(x2, flatten, ct, c2, ids)
    embed_ind = idx2d.reshape(-1)
    commit_loss = jnp.sum(val2d) / (_M * _D)

    idx_rows = embed_ind.reshape(_M // _CH, _CH)
    quantize = _make_gather_call()(codebook, idx_rows)
    quantize_st = quantize.reshape(orig_shape)
    return quantize_st, embed_ind.reshape(orig_shape[:-1]), commit_loss


# speedup vs baseline: 1.9182x; 1.0081x over previous
import functools

import jax
import jax.numpy as jnp
from jax import lax
from jax.experimental import pallas as pl
from jax.experimental.pallas import tpu as pltpu
from jax.experimental.pallas import tpu_sc as plsc

_M = 16384
_D = 256
_K = 8192

_TM = 1024
_TK = 2048
_NM = _M // _TM
_NK = _K // _TK

_NC = 2
_NS = 16
_NW = _NC * _NS
_CH = 128


def _assign_body(x2_ref, x_ref, ct_ref, c2_ref, ids_ref, idx_ref, val_ref):
    xs = x_ref[...] * 2.0
    mm = lax.dot_general(xs, ct_ref[...], (((1,), (1,)), ((), ())),
                         preferred_element_type=jnp.float32)
    t = (x2_ref[...] - mm) + c2_ref[...]
    bmin = jnp.min(t, axis=1, keepdims=True)
    barg = jnp.min(jnp.where(t == bmin, ids_ref[...], float(_K)),
                   axis=1, keepdims=True)
    idx_ref[...] = barg.astype(jnp.int32)
    val_ref[...] = bmin


_assign_call = pl.pallas_call(
    _assign_body,
    grid=(_NM,),
    in_specs=[
        pl.BlockSpec((_TM, 1), lambda m: (m, 0)),
        pl.BlockSpec((_TM, _D), lambda m: (m, 0)),
        pl.BlockSpec((_K, _D), lambda m: (0, 0)),
        pl.BlockSpec((1, _K), lambda m: (0, 0)),
        pl.BlockSpec((1, _K), lambda m: (0, 0)),
    ],
    out_specs=[
        pl.BlockSpec((_TM, 1), lambda m: (m, 0)),
        pl.BlockSpec((_TM, 1), lambda m: (m, 0)),
    ],
    out_shape=[
        jax.ShapeDtypeStruct((_M, 1), jnp.int32),
        jax.ShapeDtypeStruct((_M, 1), jnp.float32),
    ],
)


_BPW = _M // _NW
_NCH = _BPW // _CH


def _gather_body(table_hbm, idx_hbm, out_hbm, idx_v,
                 buf0, buf1, sg0, sg1, so0, so1):
    wid = lax.axis_index("s") * _NC + lax.axis_index("c")
    base = wid * _NCH
    pltpu.sync_copy(idx_hbm.at[pl.ds(base, _NCH)], idx_v)
    bufs = (buf0, buf1)
    gsems = (sg0, sg1)
    osems = (so0, so1)
    gh = {}
    oh = {}
    gh[0] = pltpu.async_copy(table_hbm.at[idx_v.at[0]], buf0, sg0)
    gh[1] = pltpu.async_copy(table_hbm.at[idx_v.at[1]], buf1, sg1)
    for j in range(_NCH):
        b = j % 2
        gh[j].wait()
        oh[j] = pltpu.async_copy(
            bufs[b], out_hbm.at[pl.ds((base + j) * _CH, _CH)], osems[b])
        if j + 2 < _NCH:
            oh[j].wait()
            gh[j + 2] = pltpu.async_copy(
                table_hbm.at[idx_v.at[j + 2]], bufs[b], gsems[b])
    oh[_NCH - 2].wait()
    oh[_NCH - 1].wait()


@functools.lru_cache(maxsize=1)
def _make_gather_call():
    return functools.partial(
        pl.kernel,
        mesh=plsc.VectorSubcoreMesh(core_axis_name="c", subcore_axis_name="s"),
        out_type=jax.ShapeDtypeStruct((_M, _D), jnp.float32),
        scratch_types=[
            pltpu.VMEM((_NCH, _CH), jnp.int32),
            pltpu.VMEM((_CH, _D), jnp.float32),
            pltpu.VMEM((_CH, _D), jnp.float32),
            pltpu.SemaphoreType.DMA,
            pltpu.SemaphoreType.DMA,
            pltpu.SemaphoreType.DMA,
            pltpu.SemaphoreType.DMA,
        ],
    )(_gather_body)


def kernel(x, codebook):
    orig_shape = x.shape
    flatten = x.reshape(-1, orig_shape[-1])
    x2 = jnp.sum(flatten ** 2, axis=1, keepdims=True)
    c2 = jnp.sum(codebook ** 2, axis=1)[None, :]

    ids = jnp.arange(_K, dtype=jnp.float32)[None, :]
    idx2d, val2d = _assign_call(x2, flatten, codebook, c2, ids)
    embed_ind = idx2d.reshape(-1)
    commit_loss = jnp.sum(val2d) / (_M * _D)

    idx_rows = embed_ind.reshape(_M // _CH, _CH)
    quantize = _make_gather_call()(codebook, idx_rows)
    quantize_st = quantize.reshape(orig_shape)
    return quantize_st, embed_ind.reshape(orig_shape[:-1]), commit_loss


# speedup vs baseline: 1.9256x; 1.0039x over previous
import functools

import jax
import jax.numpy as jnp
from jax import lax
from jax.experimental import pallas as pl
from jax.experimental.pallas import tpu as pltpu
from jax.experimental.pallas import tpu_sc as plsc

_M = 16384
_D = 256
_K = 8192

_TM = 1024
_TK = 2048
_NM = _M // _TM
_NK = _K // _TK

_NC = 2
_NS = 16
_NW = _NC * _NS
_CH = 128


def _assign_body(x2_ref, x_ref, ct_ref, c2_ref, ids_ref, idx_ref, val_ref):
    xs = x_ref[...] * 2.0
    mm = lax.dot_general(xs, ct_ref[...], (((1,), (1,)), ((), ())),
                         preferred_element_type=jnp.float32)
    t = (x2_ref[...] - mm) + c2_ref[...]
    bmin = jnp.min(t, axis=1, keepdims=True)
    barg = jnp.min(jnp.where(t == bmin, ids_ref[...], float(_K)),
                   axis=1, keepdims=True)
    idx_ref[...] = barg.astype(jnp.int32)
    val_ref[...] = bmin


_assign_call = pl.pallas_call(
    _assign_body,
    grid=(_NM,),
    in_specs=[
        pl.BlockSpec((_TM, 1), lambda m: (m, 0)),
        pl.BlockSpec((_TM, _D), lambda m: (m, 0)),
        pl.BlockSpec((_K, _D), lambda m: (0, 0)),
        pl.BlockSpec((1, _K), lambda m: (0, 0)),
        pl.BlockSpec((1, _K), lambda m: (0, 0)),
    ],
    out_specs=[
        pl.BlockSpec((_TM, 1), lambda m: (m, 0)),
        pl.BlockSpec((_TM, 1), lambda m: (m, 0)),
    ],
    out_shape=[
        jax.ShapeDtypeStruct((_M, 1), jnp.int32),
        jax.ShapeDtypeStruct((_M, 1), jnp.float32),
    ],
)


_BPW = _M // _NW
_NCH = _BPW // _CH


def _gather_body(table_hbm, idx_hbm, out_hbm, idx_v,
                 buf0, buf1, buf2, sg0, sg1, sg2, so0, so1, so2):
    wid = lax.axis_index("s") * _NC + lax.axis_index("c")
    base = wid * _NCH
    pltpu.sync_copy(idx_hbm.at[pl.ds(base, _NCH)], idx_v)
    bufs = (buf0, buf1, buf2)
    gsems = (sg0, sg1, sg2)
    osems = (so0, so1, so2)
    nb = 3
    gh = {}
    oh = {}
    for j in range(min(nb, _NCH)):
        gh[j] = pltpu.async_copy(table_hbm.at[idx_v.at[j]], bufs[j], gsems[j])
    for j in range(_NCH):
        b = j % nb
        gh[j].wait()
        oh[j] = pltpu.async_copy(
            bufs[b], out_hbm.at[pl.ds((base + j) * _CH, _CH)], osems[b])
        if j + nb < _NCH:
            oh[j].wait()
            gh[j + nb] = pltpu.async_copy(
                table_hbm.at[idx_v.at[j + nb]], bufs[b], gsems[b])
    for j in range(max(0, _NCH - nb), _NCH):
        oh[j].wait()


@functools.lru_cache(maxsize=1)
def _make_gather_call():
    return functools.partial(
        pl.kernel,
        mesh=plsc.VectorSubcoreMesh(core_axis_name="c", subcore_axis_name="s"),
        out_type=jax.ShapeDtypeStruct((_M, _D), jnp.float32),
        scratch_types=[
            pltpu.VMEM((_NCH, _CH), jnp.int32),
            pltpu.VMEM((_CH, _D), jnp.float32),
            pltpu.VMEM((_CH, _D), jnp.float32),
            pltpu.VMEM((_CH, _D), jnp.float32),
            pltpu.SemaphoreType.DMA,
            pltpu.SemaphoreType.DMA,
            pltpu.SemaphoreType.DMA,
            pltpu.SemaphoreType.DMA,
            pltpu.SemaphoreType.DMA,
            pltpu.SemaphoreType.DMA,
        ],
    )(_gather_body)


def kernel(x, codebook):
    orig_shape = x.shape
    flatten = x.reshape(-1, orig_shape[-1])
    x2 = jnp.sum(flatten ** 2, axis=1, keepdims=True)
    c2 = jnp.sum(codebook ** 2, axis=1)[None, :]

    ids = jnp.arange(_K, dtype=jnp.float32)[None, :]
    idx2d, val2d = _assign_call(x2, flatten, codebook, c2, ids)
    embed_ind = idx2d.reshape(-1)
    commit_loss = jnp.sum(val2d) / (_M * _D)

    idx_rows = embed_ind.reshape(_M // _CH, _CH)
    quantize = _make_gather_call()(codebook, idx_rows)
    quantize_st = quantize.reshape(orig_shape)
    return quantize_st, embed_ind.reshape(orig_shape[:-1]), commit_loss


# speedup vs baseline: 1.9266x; 1.0005x over previous
import functools

import jax
import jax.numpy as jnp
from jax import lax
from jax.experimental import pallas as pl
from jax.experimental.pallas import tpu as pltpu
from jax.experimental.pallas import tpu_sc as plsc

_M = 16384
_D = 256
_K = 8192

_TM = 1024
_NM = _M // _TM

_NC = 2
_NS = 16
_NW = _NC * _NS
_CH = 128


def _assign_body(x2_ref, x_ref, ct_ref, c2_ref, ids_ref, idx_ref, val_ref):
    xs = x_ref[...] * 2.0
    mm = lax.dot_general(xs, ct_ref[...], (((1,), (1,)), ((), ())),
                         preferred_element_type=jnp.float32)
    t = (x2_ref[...] - mm) + c2_ref[...]
    bmin = jnp.min(t, axis=1, keepdims=True)
    barg = jnp.min(jnp.where(t == bmin, ids_ref[...], float(_K)),
                   axis=1, keepdims=True)
    idx_ref[...] = barg.astype(jnp.int32)
    val_ref[...] = bmin


_assign_call = pl.pallas_call(
    _assign_body,
    grid=(_NM,),
    in_specs=[
        pl.BlockSpec((_TM, 1), lambda m: (m, 0)),
        pl.BlockSpec((_TM, _D), lambda m: (m, 0)),
        pl.BlockSpec((_K, _D), lambda m: (0, 0)),
        pl.BlockSpec((1, _K), lambda m: (0, 0)),
        pl.BlockSpec((1, _K), lambda m: (0, 0)),
    ],
    out_specs=[
        pl.BlockSpec((_TM, 1), lambda m: (m, 0)),
        pl.BlockSpec((_TM, 1), lambda m: (m, 0)),
    ],
    out_shape=[
        jax.ShapeDtypeStruct((_M, 1), jnp.int32),
        jax.ShapeDtypeStruct((_M, 1), jnp.float32),
    ],
)


_BPW = _M // _NW
_NCH = _BPW // _CH


def _gather_body(table_hbm, idx_hbm, out_hbm, idx_v,
                 buf0, buf1, buf2, sg0, sg1, sg2, so0, so1, so2):
    wid = lax.axis_index("s") * _NC + lax.axis_index("c")
    base = wid * _NCH
    pltpu.sync_copy(idx_hbm.at[pl.ds(base, _NCH)], idx_v)
    bufs = (buf0, buf1, buf2)
    gsems = (sg0, sg1, sg2)
    osems = (so0, so1, so2)
    nb = 3
    gh = {}
    oh = {}
    for j in range(min(nb, _NCH)):
        gh[j] = pltpu.async_copy(table_hbm.at[idx_v.at[j]], bufs[j], gsems[j])
    for j in range(_NCH):
        b = j % nb
        gh[j].wait()
        oh[j] = pltpu.async_copy(
            bufs[b], out_hbm.at[pl.ds((base + j) * _CH, _CH)], osems[b])
        if j + nb < _NCH:
            oh[j].wait()
            gh[j + nb] = pltpu.async_copy(
                table_hbm.at[idx_v.at[j + nb]], bufs[b], gsems[b])
    for j in range(max(0, _NCH - nb), _NCH):
        oh[j].wait()


@functools.lru_cache(maxsize=1)
def _make_gather_call():
    return functools.partial(
        pl.kernel,
        mesh=plsc.VectorSubcoreMesh(core_axis_name="c", subcore_axis_name="s"),
        out_type=jax.ShapeDtypeStruct((_M, _D), jnp.float32),
        scratch_types=[
            pltpu.VMEM((_NCH, _CH), jnp.int32),
            pltpu.VMEM((_CH, _D), jnp.float32),
            pltpu.VMEM((_CH, _D), jnp.float32),
            pltpu.VMEM((_CH, _D), jnp.float32),
            pltpu.SemaphoreType.DMA,
            pltpu.SemaphoreType.DMA,
            pltpu.SemaphoreType.DMA,
            pltpu.SemaphoreType.DMA,
            pltpu.SemaphoreType.DMA,
            pltpu.SemaphoreType.DMA,
        ],
    )(_gather_body)


def kernel(x, codebook):
    orig_shape = x.shape
    flatten = x.reshape(-1, orig_shape[-1])
    x2 = jnp.sum(flatten ** 2, axis=1, keepdims=True)
    c2 = jnp.sum(codebook ** 2, axis=1)[None, :]

    ids = jnp.arange(_K, dtype=jnp.float32)[None, :]
    idx2d, val2d = _assign_call(x2, flatten, codebook, c2, ids)
    embed_ind = idx2d.reshape(-1)
    commit_loss = jnp.sum(val2d) / (_M * _D)

    idx_rows = embed_ind.reshape(_M // _CH, _CH)
    quantize = _make_gather_call()(codebook, idx_rows)
    quantize_st = quantize.reshape(orig_shape)
    return quantize_st, embed_ind.reshape(orig_shape[:-1]), commit_loss
